# baseline (speedup 1.0000x reference)
import functools

import jax
import jax.numpy as jnp
from jax import lax
from jax.experimental import pallas as pl
from jax.experimental.pallas import tpu as pltpu
from jax.experimental.pallas import tpu_sc as plsc

N = 10000
D = 128
H = 64
T = 64
E = 640000

K = 128
NROWS = E // K
ROWS_PER_TILE = 156
NREM = NROWS - 32 * ROWS_PER_TILE
ICH = 26
NBUF = 9
LOOK = 7
ZROWS = 125
RPT = N // 16

_BLK = 2000


def _head_body(x_ref, w_ref, b_ref, o_ref):
    o_ref[...] = jnp.maximum(
        jnp.dot(x_ref[...], w_ref[...], preferred_element_type=jnp.float32)
        + b_ref[...], 0.0)


def _tail_body(h_ref, p0_ref, p1_ref, w1_ref, b1_ref, w2_ref, b2_ref,
               w3_ref, b3_ref, o_ref):
    m = h_ref[...] + p0_ref[0] + p1_ref[0]
    a = jnp.maximum(
        jnp.dot(m, w1_ref[...], preferred_element_type=jnp.float32)
        + b1_ref[...], 0.0)
    b = jnp.dot(a, w2_ref[...], preferred_element_type=jnp.float32) + b2_ref[...]
    o_ref[...] = jnp.dot(b, w3_ref[...], preferred_element_type=jnp.float32) + b3_ref[...]


def _sc_scatter(h_hbm, edges_hbm, zeros_hbm, out_hbm,
                srcA, dstA, srcB, dstB, rows, agg, *sems):
    gsems = sems[:NBUF]
    ssems = sems[NBUF:2 * NBUF]
    isemA, isemB = sems[2 * NBUF], sems[2 * NBUF + 1]
    c = lax.axis_index("c")
    s = lax.axis_index("s")

    pltpu.sync_copy(zeros_hbm.at[pl.ds(s * RPT, RPT)],
                    agg.at[pl.ds(s * RPT, RPT)])
    plsc.subcore_barrier()

    tile_row0 = (c * 16 + s) * ROWS_PER_TILE
    idxbuf = ((srcA, dstA, isemA), (srcB, dstB, isemB))

    def istart(chunk):
        src, dst, sem = idxbuf[chunk % 2]
        base = tile_row0 + chunk * ICH
        pltpu.make_async_copy(edges_hbm.at[0, pl.ds(base, ICH)], src, sem).start()
        pltpu.make_async_copy(edges_hbm.at[1, pl.ds(base, ICH)], dst, sem).start()

    def iwait(chunk):
        src, dst, sem = idxbuf[chunk % 2]
        pltpu.make_async_copy(edges_hbm.at[0, pl.ds(tile_row0, ICH)], src, sem).wait()
        pltpu.make_async_copy(edges_hbm.at[0, pl.ds(tile_row0, ICH)], dst, sem).wait()

    def idxrow(j):
        src, dst, _ = idxbuf[(j // ICH) % 2]
        return src.at[j % ICH], dst.at[j % ICH]

    def gstart(j):
        sref, _ = idxrow(j)
        slot = j % NBUF
        pltpu.make_async_copy(h_hbm.at[sref], rows.at[slot], gsems[slot]).start()

    def gwait(j):
        slot = j % NBUF
        pltpu.make_async_copy(h_hbm.at[srcA.at[0]], rows.at[slot],
                              gsems[slot]).wait()

    def sstart(j):
        _, dref = idxrow(j)
        slot = j % NBUF
        pltpu.async_copy(rows.at[slot], agg.at[dref], ssems[slot], add=True)

    def swait(j):
        _, dref = idxrow(j)
        slot = j % NBUF
        pltpu.make_async_copy(rows.at[slot], agg.at[dref], ssems[slot]).wait()

    istart(0)
    istart(1)
    iwait(0)
    for j in range(LOOK):
        gstart(j)
    for j in range(ROWS_PER_TILE):
        if j % ICH == NBUF - LOOK and 1 <= j // ICH < ROWS_PER_TILE // ICH - 1:
            istart(j // ICH + 1)
        jl = j + LOOK
        gwait(j)
        sstart(j)
        if jl < ROWS_PER_TILE:
            if jl - NBUF >= 0:
                swait(jl - NBUF)
            if jl % ICH == 0:
                iwait(jl // ICH)
            gstart(jl)
    for j in range(ROWS_PER_TILE - NBUF, ROWS_PER_TILE):
        swait(j)

    tid = c * 16 + s

    @pl.when(tid < NREM)
    def _():
        base = 32 * ROWS_PER_TILE + tid
        pltpu.sync_copy(edges_hbm.at[0, pl.ds(base, 1)], srcA.at[pl.ds(0, 1)])
        pltpu.sync_copy(edges_hbm.at[1, pl.ds(base, 1)], dstA.at[pl.ds(0, 1)])
        pltpu.async_copy(h_hbm.at[srcA.at[0]], rows.at[0], gsems[0]).wait()
        pltpu.sync_copy(rows.at[0], agg.at[dstA.at[0]], add=True)

    plsc.subcore_barrier()
    pltpu.sync_copy(agg.at[pl.ds(s * RPT, RPT)],
                    out_hbm.at[c, pl.ds(s * RPT, RPT)])


@functools.lru_cache(maxsize=1)
def _sc_scatter_call():
    return pl.kernel(
        _sc_scatter,
        mesh=plsc.VectorSubcoreMesh(core_axis_name="c", subcore_axis_name="s"),
        out_type=jax.ShapeDtypeStruct((2, N, H), jnp.float32),
        scratch_types=[
            pltpu.VMEM((ICH, K), jnp.int32),
            pltpu.VMEM((ICH, K), jnp.int32),
            pltpu.VMEM((ICH, K), jnp.int32),
            pltpu.VMEM((ICH, K), jnp.int32),
            pltpu.VMEM((NBUF, K, H), jnp.float32),
            pltpu.VMEM_SHARED((N, H), jnp.float32),
        ] + [pltpu.SemaphoreType.DMA] * (2 * NBUF + 2),
        compiler_params=pltpu.CompilerParams(use_tc_tiling_on_sc=False),
    )


def kernel(feature, edge_index, W0, b0, W1, b1, W2, b2, W3, b3):
    h = pl.pallas_call(
        _head_body,
        grid=(N // _BLK,),
        in_specs=[
            pl.BlockSpec((_BLK, D), lambda i: (i, 0)),
            pl.BlockSpec((D, H), lambda i: (0, 0)),
            pl.BlockSpec((1, H), lambda i: (0, 0)),
        ],
        out_specs=pl.BlockSpec((_BLK, H), lambda i: (i, 0)),
        out_shape=jax.ShapeDtypeStruct((N, H), jnp.float32),
    )(feature, W0, b0.reshape(1, H))

    edges_resh = edge_index.reshape(2, NROWS, K)
    zeros = jnp.zeros((N, H), jnp.float32)

    parts = _sc_scatter_call()(h, edges_resh, zeros)

    out = pl.pallas_call(
        _tail_body,
        grid=(N // _BLK,),
        in_specs=[
            pl.BlockSpec((_BLK, H), lambda i: (i, 0)),
            pl.BlockSpec((1, _BLK, H), lambda i: (0, i, 0)),
            pl.BlockSpec((1, _BLK, H), lambda i: (1, i, 0)),
            pl.BlockSpec((H, H), lambda i: (0, 0)),
            pl.BlockSpec((1, H), lambda i: (0, 0)),
            pl.BlockSpec((H, H), lambda i: (0, 0)),
            pl.BlockSpec((1, H), lambda i: (0, 0)),
            pl.BlockSpec((H, T), lambda i: (0, 0)),
            pl.BlockSpec((1, T), lambda i: (0, 0)),
        ],
        out_specs=pl.BlockSpec((_BLK, T), lambda i: (i, 0)),
        out_shape=jax.ShapeDtypeStruct((N, T), jnp.float32),
    )(h, parts, parts, W1, b1.reshape(1, H), W2, b2.reshape(1, H),
      W3, b3.reshape(1, T))
    return out


# speedup vs baseline: 1.0320x; 1.0320x over previous
import functools

import jax
import jax.numpy as jnp
from jax import lax
from jax.experimental import pallas as pl
from jax.experimental.pallas import tpu as pltpu
from jax.experimental.pallas import tpu_sc as plsc

N = 10000
D = 128
H = 64
T = 64
E = 640000

K = 128
NROWS = E // K
ROWS_PER_TILE = 156
NREM = NROWS - 32 * ROWS_PER_TILE
ICH = 26
NBUF = 9
LOOK = 7
ZROWS = 125
RPT = N // 16

_BLK = 5000


def _head_body(x_ref, w_ref, b_ref, o_ref):
    o_ref[...] = jnp.maximum(
        jnp.dot(x_ref[...], w_ref[...], preferred_element_type=jnp.float32)
        + b_ref[...], 0.0)


def _tail_body(h_ref, p0_ref, p1_ref, w1_ref, b1_ref, w2_ref, b2_ref,
               w3_ref, b3_ref, o_ref):
    m = h_ref[...] + p0_ref[0] + p1_ref[0]
    a = jnp.maximum(
        jnp.dot(m, w1_ref[...], preferred_element_type=jnp.float32)
        + b1_ref[...], 0.0)
    w23 = jnp.dot(w2_ref[...], w3_ref[...], preferred_element_type=jnp.float32)
    b23 = jnp.dot(b2_ref[...], w3_ref[...], preferred_element_type=jnp.float32) + b3_ref[...]
    o_ref[...] = jnp.dot(a, w23, preferred_element_type=jnp.float32) + b23


def _sc_scatter(h_hbm, edges_hbm, out_hbm,
                srcA, dstA, srcB, dstB, rows, agg, *sems):
    gsems = sems[:NBUF]
    ssems = sems[NBUF:2 * NBUF]
    isemA, isemB = sems[2 * NBUF], sems[2 * NBUF + 1]
    c = lax.axis_index("c")
    s = lax.axis_index("s")

    zb = rows.at[0]
    for r in range(ZROWS):
        for c4 in range(H // 16):
            zb[r, pl.ds(c4 * 16, 16)] = jnp.zeros((16,), jnp.float32)
    for z in range(RPT // ZROWS):
        pltpu.sync_copy(zb.at[pl.ds(0, ZROWS)],
                        agg.at[pl.ds(s * RPT + z * ZROWS, ZROWS)])
    plsc.subcore_barrier()

    tile_row0 = (c * 16 + s) * ROWS_PER_TILE
    idxbuf = ((srcA, dstA, isemA), (srcB, dstB, isemB))

    def istart(chunk):
        src, dst, sem = idxbuf[chunk % 2]
        base = tile_row0 + chunk * ICH
        pltpu.make_async_copy(edges_hbm.at[0, pl.ds(base, ICH)], src, sem).start()
        pltpu.make_async_copy(edges_hbm.at[1, pl.ds(base, ICH)], dst, sem).start()

    def iwait(chunk):
        src, dst, sem = idxbuf[chunk % 2]
        pltpu.make_async_copy(edges_hbm.at[0, pl.ds(tile_row0, ICH)], src, sem).wait()
        pltpu.make_async_copy(edges_hbm.at[0, pl.ds(tile_row0, ICH)], dst, sem).wait()

    def idxrow(j):
        src, dst, _ = idxbuf[(j // ICH) % 2]
        return src.at[j % ICH], dst.at[j % ICH]

    def gstart(j):
        sref, _ = idxrow(j)
        slot = j % NBUF
        pltpu.make_async_copy(h_hbm.at[sref], rows.at[slot], gsems[slot]).start()

    def gwait(j):
        slot = j % NBUF
        pltpu.make_async_copy(h_hbm.at[srcA.at[0]], rows.at[slot],
                              gsems[slot]).wait()

    def sstart(j):
        _, dref = idxrow(j)
        slot = j % NBUF
        pltpu.async_copy(rows.at[slot], agg.at[dref], ssems[slot], add=True)

    def swait(j):
        _, dref = idxrow(j)
        slot = j % NBUF
        pltpu.make_async_copy(rows.at[slot], agg.at[dref], ssems[slot]).wait()

    istart(0)
    istart(1)
    iwait(0)
    for j in range(LOOK):
        gstart(j)
    for j in range(ROWS_PER_TILE):
        if j % ICH == NBUF - LOOK and 1 <= j // ICH < ROWS_PER_TILE // ICH - 1:
            istart(j // ICH + 1)
        jl = j + LOOK
        gwait(j)
        sstart(j)
        if jl < ROWS_PER_TILE:
            if jl - NBUF >= 0:
                swait(jl - NBUF)
            if jl % ICH == 0:
                iwait(jl // ICH)
            gstart(jl)
    for j in range(ROWS_PER_TILE - NBUF, ROWS_PER_TILE):
        swait(j)

    tid = c * 16 + s

    @pl.when(tid < NREM)
    def _():
        base = 32 * ROWS_PER_TILE + tid
        pltpu.sync_copy(edges_hbm.at[0, pl.ds(base, 1)], srcA.at[pl.ds(0, 1)])
        pltpu.sync_copy(edges_hbm.at[1, pl.ds(base, 1)], dstA.at[pl.ds(0, 1)])
        pltpu.async_copy(h_hbm.at[srcA.at[0]], rows.at[0], gsems[0]).wait()
        pltpu.sync_copy(rows.at[0], agg.at[dstA.at[0]], add=True)

    plsc.subcore_barrier()
    pltpu.sync_copy(agg.at[pl.ds(s * RPT, RPT)],
                    out_hbm.at[c, pl.ds(s * RPT, RPT)])


@functools.lru_cache(maxsize=1)
def _sc_scatter_call():
    return pl.kernel(
        _sc_scatter,
        mesh=plsc.VectorSubcoreMesh(core_axis_name="c", subcore_axis_name="s"),
        out_type=jax.ShapeDtypeStruct((2, N, H), jnp.float32),
        scratch_types=[
            pltpu.VMEM((ICH, K), jnp.int32),
            pltpu.VMEM((ICH, K), jnp.int32),
            pltpu.VMEM((ICH, K), jnp.int32),
            pltpu.VMEM((ICH, K), jnp.int32),
            pltpu.VMEM((NBUF, K, H), jnp.float32),
            pltpu.VMEM_SHARED((N, H), jnp.float32),
        ] + [pltpu.SemaphoreType.DMA] * (2 * NBUF + 2),
        compiler_params=pltpu.CompilerParams(use_tc_tiling_on_sc=False),
    )


def kernel(feature, edge_index, W0, b0, W1, b1, W2, b2, W3, b3):
    h = pl.pallas_call(
        _head_body,
        grid=(N // _BLK,),
        in_specs=[
            pl.BlockSpec((_BLK, D), lambda i: (i, 0)),
            pl.BlockSpec((D, H), lambda i: (0, 0)),
            pl.BlockSpec((1, H), lambda i: (0, 0)),
        ],
        out_specs=pl.BlockSpec((_BLK, H), lambda i: (i, 0)),
        out_shape=jax.ShapeDtypeStruct((N, H), jnp.float32),
    )(feature, W0, b0.reshape(1, H))

    edges_resh = edge_index.reshape(2, NROWS, K)

    parts = _sc_scatter_call()(h, edges_resh)

    out = pl.pallas_call(
        _tail_body,
        grid=(N // _BLK,),
        in_specs=[
            pl.BlockSpec((_BLK, H), lambda i: (i, 0)),
            pl.BlockSpec((1, _BLK, H), lambda i: (0, i, 0)),
            pl.BlockSpec((1, _BLK, H), lambda i: (1, i, 0)),
            pl.BlockSpec((H, H), lambda i: (0, 0)),
            pl.BlockSpec((1, H), lambda i: (0, 0)),
            pl.BlockSpec((H, H), lambda i: (0, 0)),
            pl.BlockSpec((1, H), lambda i: (0, 0)),
            pl.BlockSpec((H, T), lambda i: (0, 0)),
            pl.BlockSpec((1, T), lambda i: (0, 0)),
        ],
        out_specs=pl.BlockSpec((_BLK, T), lambda i: (i, 0)),
        out_shape=jax.ShapeDtypeStruct((N, T), jnp.float32),
    )(h, parts, parts, W1, b1.reshape(1, H), W2, b2.reshape(1, H),
      W3, b3.reshape(1, T))
    return out


# speedup vs baseline: 1.0602x; 1.0273x over previous
import functools

import jax
import jax.numpy as jnp
from jax import lax
from jax.experimental import pallas as pl
from jax.experimental.pallas import tpu as pltpu
from jax.experimental.pallas import tpu_sc as plsc

N = 10000
D = 128
H = 64
T = 64
E = 640000

K = 128
NROWS = E // K
ROWS_PER_TILE = 156
NREM = NROWS - 32 * ROWS_PER_TILE
ICH = 26
NBUF = 9
LOOK = 7
ZROWS = 128
NPAD = 10240
RPT = NPAD // 16
GPT = RPT // 8

_BLK = 5000


def _head_body(x_ref, w_ref, b_ref, o_ref):
    o_ref[...] = jnp.maximum(
        jnp.dot(x_ref[...], w_ref[...], preferred_element_type=jnp.float32)
        + b_ref[...], 0.0)


def _tail_body(h_ref, p0_ref, p1_ref, w1_ref, b1_ref, w2_ref, b2_ref,
               w3_ref, b3_ref, o_ref):
    p0 = p0_ref[0, :, :, :H].reshape(_BLK, H)
    p1 = p1_ref[0, :, :, :H].reshape(_BLK, H)
    m = h_ref[...] + p0 + p1
    a = jnp.maximum(
        jnp.dot(m, w1_ref[...], preferred_element_type=jnp.float32)
        + b1_ref[...], 0.0)
    w23 = jnp.dot(w2_ref[...], w3_ref[...], preferred_element_type=jnp.float32)
    b23 = jnp.dot(b2_ref[...], w3_ref[...], preferred_element_type=jnp.float32) + b3_ref[...]
    o_ref[...] = jnp.dot(a, w23, preferred_element_type=jnp.float32) + b23


def _sc_scatter(h_hbm, edges_hbm, out_hbm,
                srcA, dstA, srcB, dstB, rows, agg, *sems):
    gsems = sems[:NBUF]
    ssems = sems[NBUF:2 * NBUF]
    isemA, isemB = sems[2 * NBUF], sems[2 * NBUF + 1]
    c = lax.axis_index("c")
    s = lax.axis_index("s")

    zb = rows.at[0]
    for r in range(ZROWS):
        for c4 in range(H // 16):
            zb[r, pl.ds(c4 * 16, 16)] = jnp.zeros((16,), jnp.float32)
    for z in range(RPT // ZROWS):
        pltpu.sync_copy(zb.at[pl.ds(0, ZROWS)],
                        agg.at[pl.ds(s * RPT + z * ZROWS, ZROWS)])
    plsc.subcore_barrier()

    tile_row0 = (c * 16 + s) * ROWS_PER_TILE
    idxbuf = ((srcA, dstA, isemA), (srcB, dstB, isemB))

    def istart(chunk):
        src, dst, sem = idxbuf[chunk % 2]
        base = tile_row0 + chunk * ICH
        pltpu.make_async_copy(edges_hbm.at[0, pl.ds(base, ICH)], src, sem).start()
        pltpu.make_async_copy(edges_hbm.at[1, pl.ds(base, ICH)], dst, sem).start()

    def iwait(chunk):
        src, dst, sem = idxbuf[chunk % 2]
        pltpu.make_async_copy(edges_hbm.at[0, pl.ds(tile_row0, ICH)], src, sem).wait()
        pltpu.make_async_copy(edges_hbm.at[0, pl.ds(tile_row0, ICH)], dst, sem).wait()

    def idxrow(j):
        src, dst, _ = idxbuf[(j // ICH) % 2]
        return src.at[j % ICH], dst.at[j % ICH]

    def gstart(j):
        sref, _ = idxrow(j)
        slot = j % NBUF
        pltpu.make_async_copy(h_hbm.at[sref], rows.at[slot], gsems[slot]).start()

    def gwait(j):
        slot = j % NBUF
        pltpu.make_async_copy(h_hbm.at[srcA.at[0]], rows.at[slot],
                              gsems[slot]).wait()

    def sstart(j):
        _, dref = idxrow(j)
        slot = j % NBUF
        pltpu.async_copy(rows.at[slot], agg.at[dref], ssems[slot], add=True)

    def swait(j):
        _, dref = idxrow(j)
        slot = j % NBUF
        pltpu.make_async_copy(rows.at[slot], agg.at[dref], ssems[slot]).wait()

    istart(0)
    istart(1)
    iwait(0)
    for j in range(LOOK):
        gstart(j)
    for j in range(ROWS_PER_TILE):
        if j % ICH == NBUF - LOOK and 1 <= j // ICH < ROWS_PER_TILE // ICH - 1:
            istart(j // ICH + 1)
        jl = j + LOOK
        gwait(j)
        sstart(j)
        if jl < ROWS_PER_TILE:
            if jl - NBUF >= 0:
                swait(jl - NBUF)
            if jl % ICH == 0:
                iwait(jl // ICH)
            gstart(jl)
    for j in range(ROWS_PER_TILE - NBUF, ROWS_PER_TILE):
        swait(j)

    tid = c * 16 + s

    @pl.when(tid < NREM)
    def _():
        base = 32 * ROWS_PER_TILE + tid
        pltpu.sync_copy(edges_hbm.at[0, pl.ds(base, 1)], srcA.at[pl.ds(0, 1)])
        pltpu.sync_copy(edges_hbm.at[1, pl.ds(base, 1)], dstA.at[pl.ds(0, 1)])
        pltpu.async_copy(h_hbm.at[srcA.at[0]], rows.at[0], gsems[0]).wait()
        pltpu.sync_copy(rows.at[0], agg.at[dstA.at[0]], add=True)

    plsc.subcore_barrier()
    g0 = s * GPT
    for gb in range(GPT // 8):
        for gg in range(8):
            g = g0 + gb * 8 + gg
            pltpu.make_async_copy(
                agg.at[pl.ds(g * 8, 8)],
                out_hbm.at[c, g, pl.ds(0, 8), pl.ds(0, H)],
                gsems[gg]).start()
        for gg in range(8):
            g = g0 + gb * 8 + gg
            pltpu.make_async_copy(
                agg.at[pl.ds(g * 8, 8)],
                out_hbm.at[c, g, pl.ds(0, 8), pl.ds(0, H)],
                gsems[gg]).wait()


@functools.lru_cache(maxsize=1)
def _sc_scatter_call():
    return pl.kernel(
        _sc_scatter,
        mesh=plsc.VectorSubcoreMesh(core_axis_name="c", subcore_axis_name="s"),
        out_type=jax.ShapeDtypeStruct((2, NPAD // 8, 8, 2 * H), jnp.float32),
        scratch_types=[
            pltpu.VMEM((ICH, K), jnp.int32),
            pltpu.VMEM((ICH, K), jnp.int32),
            pltpu.VMEM((ICH, K), jnp.int32),
            pltpu.VMEM((ICH, K), jnp.int32),
            pltpu.VMEM((NBUF, K, H), jnp.float32),
            pltpu.VMEM_SHARED((NPAD, H), jnp.float32),
        ] + [pltpu.SemaphoreType.DMA] * (2 * NBUF + 2),
        compiler_params=pltpu.CompilerParams(use_tc_tiling_on_sc=False),
    )


def kernel(feature, edge_index, W0, b0, W1, b1, W2, b2, W3, b3):
    h = pl.pallas_call(
        _head_body,
        grid=(N // _BLK,),
        in_specs=[
            pl.BlockSpec((_BLK, D), lambda i: (i, 0)),
            pl.BlockSpec((D, H), lambda i: (0, 0)),
            pl.BlockSpec((1, H), lambda i: (0, 0)),
        ],
        out_specs=pl.BlockSpec((_BLK, H), lambda i: (i, 0)),
        out_shape=jax.ShapeDtypeStruct((N, H), jnp.float32),
    )(feature, W0, b0.reshape(1, H))

    edges_resh = edge_index.reshape(2, NROWS, K)

    parts = _sc_scatter_call()(h, edges_resh)

    out = pl.pallas_call(
        _tail_body,
        grid=(N // _BLK,),
        in_specs=[
            pl.BlockSpec((_BLK, H), lambda i: (i, 0)),
            pl.BlockSpec((1, _BLK // 8, 8, 2 * H), lambda i: (0, i, 0, 0)),
            pl.BlockSpec((1, _BLK // 8, 8, 2 * H), lambda i: (1, i, 0, 0)),
            pl.BlockSpec((H, H), lambda i: (0, 0)),
            pl.BlockSpec((1, H), lambda i: (0, 0)),
            pl.BlockSpec((H, H), lambda i: (0, 0)),
            pl.BlockSpec((1, H), lambda i: (0, 0)),
            pl.BlockSpec((H, T), lambda i: (0, 0)),
            pl.BlockSpec((1, T), lambda i: (0, 0)),
        ],
        out_specs=pl.BlockSpec((_BLK, T), lambda i: (i, 0)),
        out_shape=jax.ShapeDtypeStruct((N, T), jnp.float32),
    )(h, parts, parts, W1, b1.reshape(1, H), W2, b2.reshape(1, H),
      W3, b3.reshape(1, T))
    return out
